# baseline (speedup 1.0000x reference)
import functools

import jax
import jax.numpy as jnp
from jax import lax
from jax.experimental import pallas as pl
from jax.experimental.pallas import tpu as pltpu
from jax.experimental.pallas import tpu_sc as plsc

T = 2048
D = 768
E = 8
F = 3072
K = 2
A = T * K
BLK = 256
NB = A // BLK + E
NPAD = NB * BLK
PCH = 512

NW = 32
GCH = 96
CCH = 64


def _fiota(shape, dim):
    return lax.broadcasted_iota(jnp.int32, shape, dim).astype(jnp.float32)


def _router_body(x_ref, wg_ref, bg_ref, pos_ref, tok_ref, gate_ref, be_ref):
    xf = x_ref[...]
    logits = lax.dot_general(
        wg_ref[...], xf, (((0,), (1,)), ((), ())),
        preferred_element_type=jnp.float32) + bg_ref[...]
    m = jnp.max(logits, axis=0, keepdims=True)
    ex = jnp.exp(logits - m)
    gates = ex / jnp.sum(ex, axis=0, keepdims=True)

    erow = _fiota( (E, T), 0)
    g1 = jnp.max(gates, axis=0, keepdims=True)
    i1 = jnp.min(jnp.where(gates == g1, erow, jnp.float32(E)),
                 axis=0, keepdims=True)
    gates2 = jnp.where(erow == i1, -jnp.inf, gates)
    g2 = jnp.max(gates2, axis=0, keepdims=True)
    i2 = jnp.min(jnp.where(gates2 == g2, erow, jnp.float32(E)),
                 axis=0, keepdims=True)

    ea = jnp.concatenate([i1, i2], axis=1)
    gj = jnp.concatenate([g1, g2], axis=1)

    erowA = _fiota( (E, A), 0)
    oh = (ea == erowA).astype(jnp.float32)
    cs = oh
    sh = 1
    while sh < A:
        cs = cs + jnp.concatenate(
            [jnp.zeros((E, sh), jnp.float32), cs[:, :-sh]], axis=1)
        sh *= 2
    counts = cs[:, A - 1:A]
    nblk = jnp.floor((counts + (BLK - 1)) * (1.0 / BLK))
    padded = nblk * BLK
    ltri = (_fiota( (E, E), 0)
            > _fiota( (E, E), 1)).astype(jnp.float32)
    off = jnp.dot(ltri, padded, preferred_element_type=jnp.float32)
    ends_blk = (off + padded) * (1.0 / BLK)

    posf = jnp.sum(oh * (off + cs - 1.0), axis=0, keepdims=True)
    pos_ref[...] = posf.astype(jnp.int32)

    bcol = _fiota( (1, NB), 1)
    be = jnp.sum((bcol >= ends_blk).astype(jnp.float32), axis=0, keepdims=True)
    be_ref[...] = jnp.minimum(be, jnp.float32(E - 1)).astype(jnp.int32)

    tokj = jnp.concatenate(
        [_fiota( (1, T), 1)] * 2, axis=1)
    for c in range(NPAD // PCH):
        pcol = _fiota( (PCH, 1), 0) + (c * PCH)
        mask = (posf == pcol).astype(jnp.float32)
        tok_ref[c * PCH:(c + 1) * PCH, :] = (
            jnp.sum(mask * tokj, axis=1, keepdims=True).astype(jnp.int32))
        gate_ref[c * PCH:(c + 1) * PCH, :] = (
            jnp.sum(mask * gj, axis=1, keepdims=True))


def _router_call(xf, Wg, bg):
    return pl.pallas_call(
        _router_body,
        out_shape=(
            jax.ShapeDtypeStruct((1, A), jnp.int32),
            jax.ShapeDtypeStruct((NPAD, 1), jnp.int32),
            jax.ShapeDtypeStruct((NPAD, 1), jnp.float32),
            jax.ShapeDtypeStruct((1, NB), jnp.int32),
        ),
    )(xf, Wg, bg.reshape(E, 1))


@functools.lru_cache(maxsize=None)
def _sc_dispatch():
    mesh = plsc.VectorSubcoreMesh(core_axis_name="c", subcore_axis_name="s")
    rows_per_w = NPAD // NW

    @functools.partial(
        pl.kernel,
        mesh=mesh,
        out_type=jax.ShapeDtypeStruct((NPAD, D // 2), jnp.uint32),
        scratch_types=[
            pltpu.VMEM((rows_per_w,), jnp.int32),
            pltpu.VMEM((rows_per_w, D // 2), jnp.uint32),
            pltpu.SemaphoreType.DMA,
        ],
    )
    def dispatch(x_hbm, tok_hbm, out_hbm, idx_v, rows_v, sem):
        wid = lax.axis_index("s") * 2 + lax.axis_index("c")
        base = wid * rows_per_w
        pltpu.sync_copy(tok_hbm.at[pl.ds(base, rows_per_w)], idx_v)
        pltpu.async_copy(x_hbm.at[idx_v], rows_v, sem).wait()
        pltpu.sync_copy(rows_v, out_hbm.at[pl.ds(base, rows_per_w)])

    return dispatch


_SQRT_HALF = 0.7071067811865476


def _gemm_body(be_ref, x_ref, w1_ref, b1_ref, w2_ref, b2_ref, g_ref, o_ref):
    xb = x_ref[...]
    w1 = w1_ref[0].astype(jnp.bfloat16)
    h = jnp.dot(xb, w1, preferred_element_type=jnp.float32) + b1_ref[0]
    h = 0.5 * h * (1.0 + lax.erf(h * _SQRT_HALF))
    w2 = w2_ref[0].astype(jnp.bfloat16)
    o = jnp.dot(h.astype(jnp.bfloat16), w2,
                preferred_element_type=jnp.float32) + b2_ref[0]
    o_ref[...] = o * g_ref[0, 0][:, None]


def _gemm_call(blk_expert, x_sorted, W1, b1, W2, b2, gate_sorted):
    grid_spec = pltpu.PrefetchScalarGridSpec(
        num_scalar_prefetch=1,
        grid=(NB,),
        in_specs=[
            pl.BlockSpec((BLK, D), lambda i, s: (i, 0)),
            pl.BlockSpec((1, D, F), lambda i, s: (s[i], 0, 0)),
            pl.BlockSpec((1, 1, F), lambda i, s: (s[i], 0, 0)),
            pl.BlockSpec((1, F, D), lambda i, s: (s[i], 0, 0)),
            pl.BlockSpec((1, 1, D), lambda i, s: (s[i], 0, 0)),
            pl.BlockSpec((1, 1, BLK), lambda i, s: (i, 0, 0)),
        ],
        out_specs=pl.BlockSpec((BLK, D), lambda i, s: (i, 0)),
    )
    return pl.pallas_call(
        _gemm_body,
        grid_spec=grid_spec,
        out_shape=jax.ShapeDtypeStruct((NPAD, D), jnp.float32),
    )(blk_expert, x_sorted, W1, b1.reshape(E, 1, F), W2, b2.reshape(E, 1, D),
      gate_sorted.reshape(NB, 1, BLK))


@functools.lru_cache(maxsize=None)
def _sc_combine():
    mesh = plsc.VectorSubcoreMesh(core_axis_name="c", subcore_axis_name="s")

    @functools.partial(
        pl.kernel,
        mesh=mesh,
        out_type=jax.ShapeDtypeStruct((T, D), jnp.float32),
        scratch_types=[
            pltpu.VMEM((CCH,), jnp.int32),
            pltpu.VMEM((CCH, D), jnp.float32),
            pltpu.VMEM((CCH, D), jnp.float32),
            pltpu.SemaphoreType.DMA,
            pltpu.SemaphoreType.DMA,
        ],
    )
    def combine(h_hbm, pos1_hbm, pos2_hbm, out_hbm, idx_v, a_v, b_v, s1, s2):
        wid = lax.axis_index("s") * 2 + lax.axis_index("c")
        base = wid * CCH
        pltpu.sync_copy(pos1_hbm.at[pl.ds(base, CCH)], idx_v)
        pltpu.async_copy(h_hbm.at[idx_v], a_v, s1).wait()
        pltpu.sync_copy(pos2_hbm.at[pl.ds(base, CCH)], idx_v)
        pltpu.async_copy(h_hbm.at[idx_v], b_v, s2).wait()

        def row_add(i, carry):
            for k in range(D // 16):
                sl = pl.ds(k * 16, 16)
                a_v[i, sl] = a_v[i, sl] + b_v[i, sl]
            return carry

        lax.fori_loop(0, CCH, row_add, 0)
        pltpu.sync_copy(a_v, out_hbm.at[pl.ds(base, CCH)])

    return combine


def kernel(x, Wg, bg, W1, b1, W2, b2):
    b, t, d = x.shape
    xf = x.reshape(T, D)
    pos, tok_sorted, gate_sorted, blk_expert = _router_call(xf, Wg, bg)
    x16 = xf.astype(jnp.bfloat16)
    x16v = lax.bitcast_convert_type(x16.reshape(T, D // 2, 2), jnp.uint32)
    xs_v = _sc_dispatch()(x16v, tok_sorted.reshape(NPAD))
    x_sorted = lax.bitcast_convert_type(xs_v, jnp.bfloat16).reshape(NPAD, D)
    H = _gemm_call(blk_expert.reshape(NB), x_sorted, W1, b1, W2, b2,
                   gate_sorted.reshape(NPAD))
    pos2d = pos.reshape(K, T)
    out = _sc_combine()(H, pos2d[0], pos2d[1])
    return out.reshape(b, t, d)


# speedup vs baseline: 1.3874x; 1.3874x over previous
import functools

import jax
import jax.numpy as jnp
from jax import lax
from jax.experimental import pallas as pl
from jax.experimental.pallas import tpu as pltpu
from jax.experimental.pallas import tpu_sc as plsc

T = 2048
D = 768
E = 8
F = 3072
K = 2
A = T * K
BLK = 256
NB = A // BLK + E
NPAD = NB * BLK
PCH = 512

NW = 32
GCH = 96
CCH = 64


def _fiota(shape, dim):
    return lax.broadcasted_iota(jnp.int32, shape, dim).astype(jnp.float32)


def _router_body(x_ref, wg_ref, bg_ref, pos_ref, tok_ref, gate_ref, be_ref):
    xf = x_ref[...]
    logits = lax.dot_general(
        wg_ref[...], xf, (((0,), (1,)), ((), ())),
        preferred_element_type=jnp.float32) + bg_ref[...]
    m = jnp.max(logits, axis=0, keepdims=True)
    ex = jnp.exp(logits - m)
    gates = ex / jnp.sum(ex, axis=0, keepdims=True)

    erow = _fiota( (E, T), 0)
    g1 = jnp.max(gates, axis=0, keepdims=True)
    i1 = jnp.min(jnp.where(gates == g1, erow, jnp.float32(E)),
                 axis=0, keepdims=True)
    gates2 = jnp.where(erow == i1, -jnp.inf, gates)
    g2 = jnp.max(gates2, axis=0, keepdims=True)
    i2 = jnp.min(jnp.where(gates2 == g2, erow, jnp.float32(E)),
                 axis=0, keepdims=True)

    ea = jnp.concatenate([i1, i2], axis=1)
    gj = jnp.concatenate([g1, g2], axis=1)

    erowA = _fiota( (E, A), 0)
    oh = (ea == erowA).astype(jnp.float32)
    cs = oh
    sh = 1
    while sh < A:
        cs = cs + jnp.concatenate(
            [jnp.zeros((E, sh), jnp.float32), cs[:, :-sh]], axis=1)
        sh *= 2
    counts = cs[:, A - 1:A]
    nblk = jnp.floor((counts + (BLK - 1)) * (1.0 / BLK))
    padded = nblk * BLK
    ltri = (_fiota( (E, E), 0)
            > _fiota( (E, E), 1)).astype(jnp.float32)
    off = jnp.dot(ltri, padded, preferred_element_type=jnp.float32)
    ends_blk = (off + padded) * (1.0 / BLK)

    posf = jnp.sum(oh * (off + cs - 1.0), axis=0, keepdims=True)
    pos_ref[...] = posf.astype(jnp.int32)

    bcol = _fiota( (1, NB), 1)
    be = jnp.sum((bcol >= ends_blk).astype(jnp.float32), axis=0, keepdims=True)
    be_ref[...] = jnp.minimum(be, jnp.float32(E - 1)).astype(jnp.int32)

    tokj = jnp.concatenate(
        [_fiota( (1, T), 1)] * 2, axis=1)
    for c in range(NPAD // PCH):
        pcol = _fiota( (PCH, 1), 0) + (c * PCH)
        mask = (posf == pcol).astype(jnp.float32)
        tok_ref[c * PCH:(c + 1) * PCH, :] = (
            jnp.sum(mask * tokj, axis=1, keepdims=True).astype(jnp.int32))
        gate_ref[c * PCH:(c + 1) * PCH, :] = (
            jnp.sum(mask * gj, axis=1, keepdims=True))


def _router_call(xf, Wg, bg):
    return pl.pallas_call(
        _router_body,
        out_shape=(
            jax.ShapeDtypeStruct((1, A), jnp.int32),
            jax.ShapeDtypeStruct((NPAD, 1), jnp.int32),
            jax.ShapeDtypeStruct((NPAD, 1), jnp.float32),
            jax.ShapeDtypeStruct((1, NB), jnp.int32),
        ),
    )(xf, Wg, bg.reshape(E, 1))


NCHAIN = 8


@functools.lru_cache(maxsize=None)
def _sc_dispatch():
    mesh = plsc.VectorSubcoreMesh(core_axis_name="c", subcore_axis_name="s")
    rows_per_w = NPAD // NW
    ch = 16
    nchunks = rows_per_w // ch
    nbuf = NCHAIN

    @functools.partial(
        pl.kernel,
        mesh=mesh,
        out_type=jax.ShapeDtypeStruct((NPAD, D), jnp.float32),
        scratch_types=[
            pltpu.VMEM((rows_per_w,), jnp.int32),
            pltpu.VMEM((nbuf, ch, D), jnp.float32),
        ] + [pltpu.SemaphoreType.DMA] * nbuf,
    )
    def dispatch(x_hbm, tok_hbm, out_hbm, idx_v, rows_v, *sems):
        wid = lax.axis_index("s") * 2 + lax.axis_index("c")
        base = wid * rows_per_w
        pltpu.sync_copy(tok_hbm.at[pl.ds(base, rows_per_w)], idx_v)
        copies = [None] * nchunks
        for step in range(nchunks):
            buf = step % nbuf
            if step >= nbuf:
                prev = step - nbuf
                copies[prev].wait()
                pltpu.sync_copy(rows_v.at[buf],
                                out_hbm.at[pl.ds(base + prev * ch, ch)])
            copies[step] = pltpu.async_copy(
                x_hbm.at[idx_v.at[pl.ds(step * ch, ch)]], rows_v.at[buf],
                sems[buf])
        for prev in range(nchunks - nbuf, nchunks):
            copies[prev].wait()
            pltpu.sync_copy(rows_v.at[prev % nbuf],
                            out_hbm.at[pl.ds(base + prev * ch, ch)])

    return dispatch


_SQRT_HALF = 0.7071067811865476


def _gemm_body(be_ref, x_ref, w1_ref, b1_ref, w2_ref, b2_ref, g_ref, o_ref):
    xb = x_ref[...].astype(jnp.bfloat16)
    w1 = w1_ref[0].astype(jnp.bfloat16)
    h = jnp.dot(xb, w1, preferred_element_type=jnp.float32) + b1_ref[0]
    h = 0.5 * h * (1.0 + lax.erf(h * _SQRT_HALF))
    w2 = w2_ref[0].astype(jnp.bfloat16)
    o = jnp.dot(h.astype(jnp.bfloat16), w2,
                preferred_element_type=jnp.float32) + b2_ref[0]
    o_ref[...] = o * g_ref[0, 0][:, None]


def _gemm_call(blk_expert, x_sorted, W1, b1, W2, b2, gate_sorted):
    grid_spec = pltpu.PrefetchScalarGridSpec(
        num_scalar_prefetch=1,
        grid=(NB,),
        in_specs=[
            pl.BlockSpec((BLK, D), lambda i, s: (i, 0)),
            pl.BlockSpec((1, D, F), lambda i, s: (s[i], 0, 0)),
            pl.BlockSpec((1, 1, F), lambda i, s: (s[i], 0, 0)),
            pl.BlockSpec((1, F, D), lambda i, s: (s[i], 0, 0)),
            pl.BlockSpec((1, 1, D), lambda i, s: (s[i], 0, 0)),
            pl.BlockSpec((1, 1, BLK), lambda i, s: (i, 0, 0)),
        ],
        out_specs=pl.BlockSpec((BLK, D), lambda i, s: (i, 0)),
    )
    return pl.pallas_call(
        _gemm_body,
        grid_spec=grid_spec,
        out_shape=jax.ShapeDtypeStruct((NPAD, D), jnp.float32),
    )(blk_expert, x_sorted, W1, b1.reshape(E, 1, F), W2, b2.reshape(E, 1, D),
      gate_sorted.reshape(NB, 1, BLK))


@functools.lru_cache(maxsize=None)
def _sc_combine():
    mesh = plsc.VectorSubcoreMesh(core_axis_name="c", subcore_axis_name="s")

    nch = 4
    ch = CCH // nch

    @functools.partial(
        pl.kernel,
        mesh=mesh,
        out_type=jax.ShapeDtypeStruct((T, D), jnp.float32),
        scratch_types=[
            pltpu.VMEM((CCH,), jnp.int32),
            pltpu.VMEM((CCH,), jnp.int32),
            pltpu.VMEM((CCH, D), jnp.float32),
            pltpu.VMEM((CCH, D), jnp.float32),
        ] + [pltpu.SemaphoreType.DMA] * (2 * nch),
    )
    def combine(h_hbm, pos1_hbm, pos2_hbm, out_hbm, i1_v, i2_v, a_v, b_v, *sems):
        wid = lax.axis_index("s") * 2 + lax.axis_index("c")
        base = wid * CCH
        pltpu.sync_copy(pos1_hbm.at[pl.ds(base, CCH)], i1_v)
        pltpu.sync_copy(pos2_hbm.at[pl.ds(base, CCH)], i2_v)
        copies = []
        for c in range(nch):
            sl = pl.ds(c * ch, ch)
            copies.append(pltpu.async_copy(
                h_hbm.at[i1_v.at[sl]], a_v.at[sl], sems[c]))
            copies.append(pltpu.async_copy(
                h_hbm.at[i2_v.at[sl]], b_v.at[sl], sems[nch + c]))
        for cp in copies:
            cp.wait()

        def row_add(i, carry):
            for k in range(D // 16):
                sl = pl.ds(k * 16, 16)
                a_v[i, sl] = a_v[i, sl] + b_v[i, sl]
            return carry

        lax.fori_loop(0, CCH, row_add, 0)
        pltpu.sync_copy(a_v, out_hbm.at[pl.ds(base, CCH)])

    return combine


def kernel(x, Wg, bg, W1, b1, W2, b2):
    b, t, d = x.shape
    xf = x.reshape(T, D)
    pos, tok_sorted, gate_sorted, blk_expert = _router_call(xf, Wg, bg)
    x_sorted = _sc_dispatch()(xf, tok_sorted.reshape(NPAD))
    H = _gemm_call(blk_expert.reshape(NB), x_sorted, W1, b1, W2, b2,
                   gate_sorted.reshape(NPAD))
    pos2d = pos.reshape(K, T)
    out = _sc_combine()(H, pos2d[0], pos2d[1])
    return out.reshape(b, t, d)


# speedup vs baseline: 1.9026x; 1.3714x over previous
import functools

import jax
import jax.numpy as jnp
from jax import lax
from jax.experimental import pallas as pl
from jax.experimental.pallas import tpu as pltpu
from jax.experimental.pallas import tpu_sc as plsc

T = 2048
D = 768
E = 8
F = 3072
K = 2
A = T * K
BLK = 256
NB = A // BLK + E
NPAD = NB * BLK
PCH = 512

NW = 32
GCH = 96
CCH = 64


def _fiota(shape, dim):
    return lax.broadcasted_iota(jnp.int32, shape, dim).astype(jnp.float32)


def _router_body(x_ref, wg_ref, bg_ref, pos_ref, tok_ref, gate_ref, be_ref,
                 x16_ref):
    xf = x_ref[...]
    x16_ref[...] = xf.astype(jnp.bfloat16)
    logits = lax.dot_general(
        wg_ref[...], xf, (((0,), (1,)), ((), ())),
        preferred_element_type=jnp.float32) + bg_ref[...]
    m = jnp.max(logits, axis=0, keepdims=True)
    ex = jnp.exp(logits - m)
    gates = ex / jnp.sum(ex, axis=0, keepdims=True)

    erow = _fiota( (E, T), 0)
    g1 = jnp.max(gates, axis=0, keepdims=True)
    i1 = jnp.min(jnp.where(gates == g1, erow, jnp.float32(E)),
                 axis=0, keepdims=True)
    gates2 = jnp.where(erow == i1, -jnp.inf, gates)
    g2 = jnp.max(gates2, axis=0, keepdims=True)
    i2 = jnp.min(jnp.where(gates2 == g2, erow, jnp.float32(E)),
                 axis=0, keepdims=True)

    ea = jnp.concatenate([i1, i2], axis=1)
    gj = jnp.concatenate([g1, g2], axis=1)

    erowA = _fiota( (E, A), 0)
    oh = (ea == erowA).astype(jnp.float32)
    cs = oh
    sh = 1
    while sh < A:
        cs = cs + jnp.concatenate(
            [jnp.zeros((E, sh), jnp.float32), cs[:, :-sh]], axis=1)
        sh *= 2
    counts = cs[:, A - 1:A]
    nblk = jnp.floor((counts + (BLK - 1)) * (1.0 / BLK))
    padded = nblk * BLK
    ltri = (_fiota( (E, E), 0)
            > _fiota( (E, E), 1)).astype(jnp.float32)
    off = jnp.dot(ltri, padded, preferred_element_type=jnp.float32)
    ends_blk = (off + padded) * (1.0 / BLK)

    posf = jnp.sum(oh * (off + cs - 1.0), axis=0, keepdims=True)
    pos_ref[...] = posf.astype(jnp.int32)

    bcol = _fiota( (1, NB), 1)
    be = jnp.sum((bcol >= ends_blk).astype(jnp.float32), axis=0, keepdims=True)
    be_ref[...] = jnp.minimum(be, jnp.float32(E - 1)).astype(jnp.int32)

    tokj = jnp.concatenate(
        [_fiota( (1, T), 1)] * 2, axis=1)
    for c in range(NPAD // PCH):
        pcol = _fiota( (PCH, 1), 0) + (c * PCH)
        mask = (posf == pcol).astype(jnp.float32)
        tok_ref[c * PCH:(c + 1) * PCH, :] = (
            jnp.sum(mask * tokj, axis=1, keepdims=True).astype(jnp.int32))
        gate_ref[c * PCH:(c + 1) * PCH, :] = (
            jnp.sum(mask * gj, axis=1, keepdims=True))


def _router_call(xf, Wg, bg):
    return pl.pallas_call(
        _router_body,
        out_shape=(
            jax.ShapeDtypeStruct((1, A), jnp.int32),
            jax.ShapeDtypeStruct((NPAD, 1), jnp.int32),
            jax.ShapeDtypeStruct((NPAD, 1), jnp.float32),
            jax.ShapeDtypeStruct((1, NB), jnp.int32),
            jax.ShapeDtypeStruct((T, D), jnp.bfloat16),
        ),
    )(xf, Wg, bg.reshape(E, 1))


_SQRT_HALF = 0.7071067811865476


def _gemm_body(be_ref, x16_ref, tok_ref, w1_ref, b1_ref, w2_ref, b2_ref,
               g_ref, o_ref):
    tok = tok_ref[0, 0]
    tcol = lax.broadcast_in_dim(tok, (BLK, T), (0,))
    sel = (lax.broadcasted_iota(jnp.int32, (BLK, T), 1) == tcol)
    p = sel.astype(jnp.bfloat16)
    xb = jnp.dot(p, x16_ref[...],
                 preferred_element_type=jnp.float32).astype(jnp.bfloat16)
    w1 = w1_ref[0].astype(jnp.bfloat16)
    h = jnp.dot(xb, w1, preferred_element_type=jnp.float32) + b1_ref[0]
    h = 0.5 * h * (1.0 + lax.erf(h * _SQRT_HALF))
    w2 = w2_ref[0].astype(jnp.bfloat16)
    o = jnp.dot(h.astype(jnp.bfloat16), w2,
                preferred_element_type=jnp.float32) + b2_ref[0]
    o_ref[...] = o * g_ref[0, 0][:, None]


def _gemm_call(blk_expert, x16, tok_sorted, W1, b1, W2, b2, gate_sorted):
    grid_spec = pltpu.PrefetchScalarGridSpec(
        num_scalar_prefetch=1,
        grid=(NB,),
        in_specs=[
            pl.BlockSpec((T, D), lambda i, s: (0, 0)),
            pl.BlockSpec((1, 1, BLK), lambda i, s: (i, 0, 0)),
            pl.BlockSpec((1, D, F), lambda i, s: (s[i], 0, 0)),
            pl.BlockSpec((1, 1, F), lambda i, s: (s[i], 0, 0)),
            pl.BlockSpec((1, F, D), lambda i, s: (s[i], 0, 0)),
            pl.BlockSpec((1, 1, D), lambda i, s: (s[i], 0, 0)),
            pl.BlockSpec((1, 1, BLK), lambda i, s: (i, 0, 0)),
        ],
        out_specs=pl.BlockSpec((BLK, D), lambda i, s: (i, 0)),
    )
    return pl.pallas_call(
        _gemm_body,
        grid_spec=grid_spec,
        out_shape=jax.ShapeDtypeStruct((NPAD, D), jnp.float32),
    )(blk_expert, x16, tok_sorted.reshape(NB, 1, BLK), W1,
      b1.reshape(E, 1, F), W2, b2.reshape(E, 1, D),
      gate_sorted.reshape(NB, 1, BLK))


@functools.lru_cache(maxsize=None)
def _sc_combine():
    mesh = plsc.VectorSubcoreMesh(core_axis_name="c", subcore_axis_name="s")

    nch = 4
    ch = CCH // nch

    @functools.partial(
        pl.kernel,
        mesh=mesh,
        out_type=jax.ShapeDtypeStruct((T, D), jnp.float32),
        scratch_types=[
            pltpu.VMEM((CCH,), jnp.int32),
            pltpu.VMEM((CCH,), jnp.int32),
            pltpu.VMEM((CCH, D), jnp.float32),
            pltpu.VMEM((CCH, D), jnp.float32),
        ] + [pltpu.SemaphoreType.DMA] * (2 * nch),
    )
    def combine(h_hbm, pos1_hbm, pos2_hbm, out_hbm, i1_v, i2_v, a_v, b_v, *sems):
        wid = lax.axis_index("s") * 2 + lax.axis_index("c")
        base = wid * CCH
        pltpu.sync_copy(pos1_hbm.at[pl.ds(base, CCH)], i1_v)
        pltpu.sync_copy(pos2_hbm.at[pl.ds(base, CCH)], i2_v)
        copies = []
        for c in range(nch):
            sl = pl.ds(c * ch, ch)
            copies.append(pltpu.async_copy(
                h_hbm.at[i1_v.at[sl]], a_v.at[sl], sems[c]))
            copies.append(pltpu.async_copy(
                h_hbm.at[i2_v.at[sl]], b_v.at[sl], sems[nch + c]))
        for cp in copies:
            cp.wait()

        def row_add(i, carry):
            for k in range(D // 16):
                sl = pl.ds(k * 16, 16)
                a_v[i, sl] = a_v[i, sl] + b_v[i, sl]
            return carry

        lax.fori_loop(0, CCH, row_add, 0)
        pltpu.sync_copy(a_v, out_hbm.at[pl.ds(base, CCH)])

    return combine


def kernel(x, Wg, bg, W1, b1, W2, b2):
    b, t, d = x.shape
    xf = x.reshape(T, D)
    pos, tok_sorted, gate_sorted, blk_expert, x16 = _router_call(xf, Wg, bg)
    H = _gemm_call(blk_expert.reshape(NB), x16, tok_sorted.reshape(NPAD),
                   W1, b1, W2, b2, gate_sorted.reshape(NPAD))
    pos2d = pos.reshape(K, T)
    out = _sc_combine()(H, pos2d[0], pos2d[1])
    return out.reshape(b, t, d)


# speedup vs baseline: 2.3585x; 1.2396x over previous
import functools

import jax
import jax.numpy as jnp
from jax import lax
from jax.experimental import pallas as pl
from jax.experimental.pallas import tpu as pltpu
from jax.experimental.pallas import tpu_sc as plsc

T = 2048
D = 768
E = 8
F = 3072
K = 2
A = T * K
BLK = 256
NB = A // BLK + E
NPAD = NB * BLK
PCH = 512

NW = 32
GCH = 96
CCH = 64


def _fiota(shape, dim):
    return lax.broadcasted_iota(jnp.int32, shape, dim).astype(jnp.float32)


def _router_body(x_ref, wg_ref, bg_ref, pos_ref, gj_ref, s_ref, x16_ref):
    xf = x_ref[...]
    x16_ref[...] = xf.astype(jnp.bfloat16)
    logits = lax.dot_general(
        wg_ref[...], xf, (((0,), (1,)), ((), ())),
        preferred_element_type=jnp.float32) + bg_ref[...]
    m = jnp.max(logits, axis=0, keepdims=True)
    ex = jnp.exp(logits - m)
    gates = ex / jnp.sum(ex, axis=0, keepdims=True)

    erow = _fiota( (E, T), 0)
    g1 = jnp.max(gates, axis=0, keepdims=True)
    i1 = jnp.min(jnp.where(gates == g1, erow, jnp.float32(E)),
                 axis=0, keepdims=True)
    gates2 = jnp.where(erow == i1, -jnp.inf, gates)
    g2 = jnp.max(gates2, axis=0, keepdims=True)
    i2 = jnp.min(jnp.where(gates2 == g2, erow, jnp.float32(E)),
                 axis=0, keepdims=True)

    ea = jnp.concatenate([i1, i2], axis=1)
    gj = jnp.concatenate([g1, g2], axis=1)

    erowA = _fiota( (E, A), 0)
    oh = (ea == erowA).astype(jnp.float32)
    cs = oh
    sh = 1
    while sh < A:
        cs = cs + jnp.concatenate(
            [jnp.zeros((E, sh), jnp.float32), cs[:, :-sh]], axis=1)
        sh *= 2
    counts = cs[:, A - 1:A]
    nblk = jnp.floor((counts + (BLK - 1)) * (1.0 / BLK))
    padded = nblk * BLK
    ltri = (_fiota( (E, E), 0)
            > _fiota( (E, E), 1)).astype(jnp.float32)
    off = jnp.dot(ltri, padded, preferred_element_type=jnp.float32)
    ends_blk = (off + padded) * (1.0 / BLK)

    posf = jnp.sum(oh * (off + cs - 1.0), axis=0, keepdims=True)
    pos_ref[...] = posf.astype(jnp.int32)
    gj_ref[...] = gj

    bcol = _fiota( (1, NB), 1)
    be = jnp.sum((bcol >= ends_blk).astype(jnp.float32), axis=0, keepdims=True)
    erow_c = _fiota( (E, 1), 0)
    emax = jnp.max(jnp.where(counts > 0.0, erow_c, 0.0), axis=0,
                   keepdims=True)
    be = jnp.minimum(be, emax)
    total = jnp.max(ends_blk, axis=0, keepdims=True)
    s_ref[...] = jnp.concatenate([be, total], axis=1).astype(jnp.int32)


def _router_call(xf, Wg, bg):
    return pl.pallas_call(
        _router_body,
        out_shape=(
            jax.ShapeDtypeStruct((1, A), jnp.int32),
            jax.ShapeDtypeStruct((1, A), jnp.float32),
            jax.ShapeDtypeStruct((1, NB + 1), jnp.int32),
            jax.ShapeDtypeStruct((T, D), jnp.bfloat16),
        ),
    )(xf, Wg, bg.reshape(E, 1))


_SQRT_HALF = 0.7071067811865476


def _gemm_body(s_ref, x16_ref, pos_ref, w1_ref, b1_ref, w2_ref, b2_ref,
               o_ref):
    i = pl.program_id(0)

    @pl.when(i < s_ref[NB])
    def _active():
        slot = lax.broadcasted_iota(jnp.int32, (BLK, T), 0) + i * BLK
        p1 = lax.broadcast_in_dim(pos_ref[0:1, :], (BLK, T), (0, 1))
        p2 = lax.broadcast_in_dim(pos_ref[1:2, :], (BLK, T), (0, 1))
        p = ((p1 == slot) | (p2 == slot)).astype(jnp.bfloat16)
        xb = jnp.dot(p, x16_ref[...],
                     preferred_element_type=jnp.float32).astype(jnp.bfloat16)
        w1 = w1_ref[0].astype(jnp.bfloat16)
        h = jnp.dot(xb, w1, preferred_element_type=jnp.float32) + b1_ref[0]
        h = 0.5 * h * (1.0 + lax.erf(h * _SQRT_HALF))
        w2 = w2_ref[0].astype(jnp.bfloat16)
        o_ref[...] = jnp.dot(h.astype(jnp.bfloat16), w2,
                             preferred_element_type=jnp.float32) + b2_ref[0]


def _gemm_call(s, x16, pos2d, W1, b1, W2, b2):
    grid_spec = pltpu.PrefetchScalarGridSpec(
        num_scalar_prefetch=1,
        grid=(NB,),
        in_specs=[
            pl.BlockSpec((T, D), lambda i, s: (0, 0)),
            pl.BlockSpec((K, T), lambda i, s: (0, 0)),
            pl.BlockSpec((1, D, F), lambda i, s: (s[i], 0, 0)),
            pl.BlockSpec((1, 1, F), lambda i, s: (s[i], 0, 0)),
            pl.BlockSpec((1, F, D), lambda i, s: (s[i], 0, 0)),
            pl.BlockSpec((1, 1, D), lambda i, s: (s[i], 0, 0)),
        ],
        out_specs=pl.BlockSpec((BLK, D), lambda i, s: (i, 0)),
    )
    return pl.pallas_call(
        _gemm_body,
        grid_spec=grid_spec,
        out_shape=jax.ShapeDtypeStruct((NPAD, D), jnp.float32),
    )(s, x16, pos2d, W1, b1.reshape(E, 1, F), W2, b2.reshape(E, 1, D))


@functools.lru_cache(maxsize=None)
def _sc_combine():
    mesh = plsc.VectorSubcoreMesh(core_axis_name="c", subcore_axis_name="s")

    nch = 4
    ch = CCH // nch

    @functools.partial(
        pl.kernel,
        mesh=mesh,
        out_type=jax.ShapeDtypeStruct((T, D), jnp.float32),
        scratch_types=[
            pltpu.VMEM((CCH,), jnp.int32),
            pltpu.VMEM((CCH,), jnp.int32),
            pltpu.VMEM((CCH + 16,), jnp.float32),
            pltpu.VMEM((CCH + 16,), jnp.float32),
            pltpu.VMEM((CCH, D), jnp.float32),
            pltpu.VMEM((CCH, D), jnp.float32),
        ] + [pltpu.SemaphoreType.DMA] * (2 * nch),
    )
    def combine(h_hbm, pos1_hbm, pos2_hbm, g1_hbm, g2_hbm, out_hbm,
                i1_v, i2_v, g1_v, g2_v, a_v, b_v, *sems):
        wid = lax.axis_index("s") * 2 + lax.axis_index("c")
        base = wid * CCH
        pltpu.sync_copy(pos1_hbm.at[pl.ds(base, CCH)], i1_v)
        pltpu.sync_copy(pos2_hbm.at[pl.ds(base, CCH)], i2_v)
        copies = []
        for c in range(nch):
            sl = pl.ds(c * ch, ch)
            copies.append(pltpu.async_copy(
                h_hbm.at[i1_v.at[sl]], a_v.at[sl], sems[c]))
            copies.append(pltpu.async_copy(
                h_hbm.at[i2_v.at[sl]], b_v.at[sl], sems[nch + c]))
        pltpu.sync_copy(g1_hbm.at[pl.ds(base, CCH)], g1_v.at[pl.ds(0, CCH)])
        pltpu.sync_copy(g2_hbm.at[pl.ds(base, CCH)], g2_v.at[pl.ds(0, CCH)])
        for cp in copies:
            cp.wait()

        def row_add(i, carry):
            ga = g1_v[pl.ds(i, 16)][0]
            gb = g2_v[pl.ds(i, 16)][0]
            for k in range(D // 16):
                sl = pl.ds(k * 16, 16)
                a_v[i, sl] = a_v[i, sl] * ga + b_v[i, sl] * gb
            return carry

        lax.fori_loop(0, CCH, row_add, 0)
        pltpu.sync_copy(a_v, out_hbm.at[pl.ds(base, CCH)])

    return combine


def kernel(x, Wg, bg, W1, b1, W2, b2):
    b, t, d = x.shape
    xf = x.reshape(T, D)
    pos, gj, s, x16 = _router_call(xf, Wg, bg)
    pos2d = pos.reshape(K, T)
    H = _gemm_call(s.reshape(NB + 1), x16, pos2d, W1, b1, W2, b2)
    gj2d = gj.reshape(K, T)
    out = _sc_combine()(H, pos2d[0], pos2d[1], gj2d[0], gj2d[1])
    return out.reshape(b, t, d)
